# Initial kernel scaffold; baseline (speedup 1.0000x reference)
import jax
import jax.numpy as jnp
from jax.experimental import pallas as pl


def kernel(ids, emb, W1, b1, W2, b2):
    raise NotImplementedError("write your pallas kernel here")


# speedup vs baseline: 1.1806x; 1.1806x over previous
import dataclasses
import functools

import jax
import jax.numpy as jnp
from jax import lax
from jax.experimental import pallas as pl
from jax.experimental.pallas import tpu as pltpu
from jax.experimental.pallas import tpu_sc as plsc

_NUM_CORES = 2
_NUM_SUBCORES = 16
_NUM_TILES = _NUM_CORES * _NUM_SUBCORES
_LANES = 16
_BINS = 256


def _sc_compiler_params():
    cp = pltpu.CompilerParams()
    if "needs_layout_passes" in pltpu.CompilerParams.__dataclass_fields__:
        cp = dataclasses.replace(cp, needs_layout_passes=False)
    return cp


def _make_sc_histogram(num_ids: int):
    per_tile = num_ids // _NUM_TILES
    mesh = plsc.VectorSubcoreMesh(
        core_axis_name="c", subcore_axis_name="s",
        num_cores=_NUM_CORES, num_subcores=_NUM_SUBCORES)

    @functools.partial(
        pl.kernel,
        out_type=jax.ShapeDtypeStruct((_NUM_TILES, _BINS), jnp.float32),
        mesh=mesh,
        scratch_types=[
            pltpu.VMEM((per_tile,), jnp.int32),
            pltpu.VMEM((_LANES * _BINS,), jnp.float32),
        ],
        compiler_params=_sc_compiler_params(),
    )
    def sc_histogram(ids_hbm, out_hbm, ids_v, hist_v):
        wid = lax.axis_index("s") * _NUM_CORES + lax.axis_index("c")
        pltpu.sync_copy(ids_hbm.at[pl.ds(wid * per_tile, per_tile)], ids_v)

        zeros = jnp.zeros((_LANES,), jnp.float32)

        @pl.loop(0, _LANES * _BINS, step=_LANES)
        def _(i):
            hist_v[pl.ds(i, _LANES)] = zeros

        lane_base = lax.iota(jnp.int32, _LANES) * _BINS
        ones = jnp.ones((_LANES,), jnp.float32)

        @pl.loop(0, per_tile, step=_LANES)
        def _(i):
            idx = lane_base + ids_v[pl.ds(i, _LANES)]
            plsc.addupdate_scatter(hist_v, [idx], ones)

        @pl.loop(0, _BINS, step=_LANES)
        def _(i):
            acc = hist_v[pl.ds(i, _LANES)]
            for j in range(1, _LANES):
                acc = acc + hist_v[pl.ds(j * _BINS + i, _LANES)]
            hist_v[pl.ds(i, _LANES)] = acc

        pltpu.sync_copy(hist_v.at[pl.ds(0, _BINS)], out_hbm.at[wid])

    return sc_histogram


def _tc_head(parts, emb, W1, b1, W2, b2, inv_l):
    def body(parts_ref, emb_ref, w1_ref, b1_ref, w2_ref, b2_ref, out_ref):
        hi = lax.Precision.HIGHEST
        counts = jnp.sum(parts_ref[...], axis=0, keepdims=True)
        e = lax.dot_general(
            counts, emb_ref[...], (((1,), (0,)), ((), ())),
            precision=hi, preferred_element_type=jnp.float32) * inv_l
        h = lax.dot_general(
            e, w1_ref[...], (((1,), (1,)), ((), ())),
            precision=hi, preferred_element_type=jnp.float32) + b1_ref[...]
        h = jnp.maximum(h, 0.0)
        v = lax.dot_general(
            h, w2_ref[...], (((1,), (1,)), ((), ())),
            precision=hi, preferred_element_type=jnp.float32) + b2_ref[...]
        out_ref[...] = v

    return pl.pallas_call(
        body,
        out_shape=jax.ShapeDtypeStruct((1, 8), jnp.float32),
    )(parts, emb, W1, b1.reshape(1, -1), W2, b2.reshape(1, -1))


def kernel(ids, emb, W1, b1, W2, b2):
    num_ids = ids.shape[0]
    parts = _make_sc_histogram(num_ids)(ids.astype(jnp.int32))
    out = _tc_head(parts, emb, W1, b1, W2, b2, 1.0 / num_ids)
    return out.reshape(8)


# speedup vs baseline: 1.2612x; 1.0683x over previous
import dataclasses
import functools

import jax
import jax.numpy as jnp
from jax import lax
from jax.experimental import pallas as pl
from jax.experimental.pallas import tpu as pltpu
from jax.experimental.pallas import tpu_sc as plsc

_NUM_CORES = 2
_NUM_SUBCORES = 16
_NUM_TILES = _NUM_CORES * _NUM_SUBCORES
_LANES = 16
_BINS = 256


def _sc_compiler_params():
    cp = pltpu.CompilerParams()
    if "needs_layout_passes" in pltpu.CompilerParams.__dataclass_fields__:
        cp = dataclasses.replace(cp, needs_layout_passes=False)
    return cp


def _make_sc_histogram(num_ids: int):
    per_tile = num_ids // _NUM_TILES
    mesh = plsc.VectorSubcoreMesh(
        core_axis_name="c", subcore_axis_name="s",
        num_cores=_NUM_CORES, num_subcores=_NUM_SUBCORES)

    @functools.partial(
        pl.kernel,
        out_type=jax.ShapeDtypeStruct((_NUM_TILES, _BINS), jnp.int32),
        mesh=mesh,
        scratch_types=[
            pltpu.VMEM((per_tile,), jnp.int32),
            pltpu.VMEM((_BINS,), jnp.int32),
        ],
        compiler_params=_sc_compiler_params(),
    )
    def sc_histogram(ids_hbm, out_hbm, ids_v, hist_v):
        wid = lax.axis_index("s") * _NUM_CORES + lax.axis_index("c")
        pltpu.sync_copy(ids_hbm.at[pl.ds(wid * per_tile, per_tile)], ids_v)

        zeros = jnp.zeros((_LANES,), jnp.int32)

        @pl.loop(0, _BINS, step=_LANES)
        def _(i):
            hist_v[pl.ds(i, _LANES)] = zeros

        @pl.loop(0, per_tile, step=_LANES)
        def _(i):
            ids_vec = ids_v[pl.ds(i, _LANES)]
            counts, last = plsc.scan_count(ids_vec)
            plsc.addupdate_scatter(hist_v, [ids_vec], counts, mask=last)

        pltpu.sync_copy(hist_v, out_hbm.at[wid])

    return sc_histogram


def _tc_head(parts, emb, W1, b1, W2, b2, inv_l):
    def body(parts_ref, emb_ref, w1_ref, b1_ref, w2_ref, b2_ref, out_ref):
        hi = lax.Precision.HIGHEST
        counts = jnp.sum(parts_ref[...].astype(jnp.float32),
                         axis=0, keepdims=True)
        e = lax.dot_general(
            counts, emb_ref[...], (((1,), (0,)), ((), ())),
            precision=hi, preferred_element_type=jnp.float32) * inv_l
        h = lax.dot_general(
            e, w1_ref[...], (((1,), (1,)), ((), ())),
            precision=hi, preferred_element_type=jnp.float32) + b1_ref[...]
        h = jnp.maximum(h, 0.0)
        v = lax.dot_general(
            h, w2_ref[...], (((1,), (1,)), ((), ())),
            precision=hi, preferred_element_type=jnp.float32) + b2_ref[...]
        out_ref[...] = v

    return pl.pallas_call(
        body,
        out_shape=jax.ShapeDtypeStruct((1, 8), jnp.float32),
    )(parts, emb, W1, b1.reshape(1, -1), W2, b2.reshape(1, -1))


def kernel(ids, emb, W1, b1, W2, b2):
    num_ids = ids.shape[0]
    parts = _make_sc_histogram(num_ids)(ids.astype(jnp.int32))
    out = _tc_head(parts, emb, W1, b1, W2, b2, 1.0 / num_ids)
    return out.reshape(8)


# speedup vs baseline: 1.2702x; 1.0072x over previous
import dataclasses
import functools

import jax
import jax.numpy as jnp
from jax import lax
from jax.experimental import pallas as pl
from jax.experimental.pallas import tpu as pltpu
from jax.experimental.pallas import tpu_sc as plsc

_NUM_CORES = 2
_NUM_SUBCORES = 16
_NUM_TILES = _NUM_CORES * _NUM_SUBCORES
_LANES = 16
_BINS = 256


def _sc_compiler_params():
    cp = pltpu.CompilerParams()
    if "needs_layout_passes" in pltpu.CompilerParams.__dataclass_fields__:
        cp = dataclasses.replace(cp, needs_layout_passes=False)
    return cp


def _make_sc_histogram(num_ids: int):
    per_tile = num_ids // _NUM_TILES
    mesh = plsc.VectorSubcoreMesh(
        core_axis_name="c", subcore_axis_name="s",
        num_cores=_NUM_CORES, num_subcores=_NUM_SUBCORES)

    @functools.partial(
        pl.kernel,
        out_type=jax.ShapeDtypeStruct((_NUM_TILES, _BINS), jnp.int32),
        mesh=mesh,
        scratch_types=[
            pltpu.VMEM((per_tile,), jnp.int32),
            pltpu.VMEM((_BINS,), jnp.int32),
        ],
        compiler_params=_sc_compiler_params(),
    )
    def sc_histogram(ids_hbm, out_hbm, ids_v, hist_v):
        wid = lax.axis_index("s") * _NUM_CORES + lax.axis_index("c")
        pltpu.sync_copy(ids_hbm.at[pl.ds(wid * per_tile, per_tile)], ids_v)

        zeros = jnp.zeros((_LANES,), jnp.int32)

        @pl.loop(0, _BINS, step=_LANES)
        def _(i):
            hist_v[pl.ds(i, _LANES)] = zeros

        @pl.loop(0, per_tile, step=_LANES)
        def _(i):
            ids_vec = ids_v[pl.ds(i, _LANES)]
            counts, last = plsc.scan_count(ids_vec)
            plsc.addupdate_scatter(hist_v, [ids_vec], counts, mask=last)

        pltpu.sync_copy(hist_v, out_hbm.at[wid])

    return sc_histogram


def _tc_head(parts, emb, W1, b1, W2, b2, inv_l):
    def body(parts_ref, emb_ref, w1_ref, b1_ref, w2_ref, b2_ref, out_ref):
        hi = lax.Precision.HIGHEST
        counts = jnp.sum(parts_ref[...].astype(jnp.float32),
                         axis=0, keepdims=True)
        e = lax.dot_general(
            counts, emb_ref[...], (((1,), (0,)), ((), ())),
            precision=hi, preferred_element_type=jnp.float32) * inv_l
        h = lax.dot_general(
            e, w1_ref[...], (((1,), (1,)), ((), ())),
            preferred_element_type=jnp.float32) + b1_ref[...]
        h = jnp.maximum(h, 0.0)
        v = lax.dot_general(
            h, w2_ref[...], (((1,), (1,)), ((), ())),
            preferred_element_type=jnp.float32) + b2_ref[...]
        out_ref[...] = v

    return pl.pallas_call(
        body,
        out_shape=jax.ShapeDtypeStruct((1, 8), jnp.float32),
    )(parts, emb, W1, b1.reshape(1, -1), W2, b2.reshape(1, -1))


def kernel(ids, emb, W1, b1, W2, b2):
    num_ids = ids.shape[0]
    parts = _make_sc_histogram(num_ids)(ids.astype(jnp.int32))
    out = _tc_head(parts, emb, W1, b1, W2, b2, 1.0 / num_ids)
    return out.reshape(8)


# speedup vs baseline: 1.3576x; 1.0688x over previous
import dataclasses
import functools

import jax
import jax.numpy as jnp
from jax import lax
from jax.experimental import pallas as pl
from jax.experimental.pallas import tpu as pltpu
from jax.experimental.pallas import tpu_sc as plsc

_NUM_CORES = 1
_NUM_SUBCORES = 16
_NUM_TILES = _NUM_CORES * _NUM_SUBCORES
_LANES = 16
_BINS = 256


def _sc_compiler_params():
    cp = pltpu.CompilerParams()
    if "needs_layout_passes" in pltpu.CompilerParams.__dataclass_fields__:
        cp = dataclasses.replace(cp, needs_layout_passes=False)
    return cp


def _make_sc_histogram(num_ids: int):
    per_tile = num_ids // _NUM_TILES
    mesh = plsc.VectorSubcoreMesh(
        core_axis_name="c", subcore_axis_name="s",
        num_cores=_NUM_CORES, num_subcores=_NUM_SUBCORES)

    @functools.partial(
        pl.kernel,
        out_type=jax.ShapeDtypeStruct((_NUM_TILES, _BINS), jnp.int32),
        mesh=mesh,
        scratch_types=[
            pltpu.VMEM((per_tile,), jnp.int32),
            pltpu.VMEM((_BINS,), jnp.int32),
        ],
        compiler_params=_sc_compiler_params(),
    )
    def sc_histogram(ids_hbm, out_hbm, ids_v, hist_v):
        wid = lax.axis_index("s") * _NUM_CORES + lax.axis_index("c")
        pltpu.sync_copy(ids_hbm.at[pl.ds(wid * per_tile, per_tile)], ids_v)

        zeros = jnp.zeros((_LANES,), jnp.int32)

        @pl.loop(0, _BINS, step=_LANES)
        def _(i):
            hist_v[pl.ds(i, _LANES)] = zeros

        @pl.loop(0, per_tile, step=_LANES)
        def _(i):
            ids_vec = ids_v[pl.ds(i, _LANES)]
            counts, last = plsc.scan_count(ids_vec)
            plsc.addupdate_scatter(hist_v, [ids_vec], counts, mask=last)

        pltpu.sync_copy(hist_v, out_hbm.at[wid])

    return sc_histogram


def _tc_head(parts, emb, W1, b1, W2, b2, inv_l):
    def body(parts_ref, emb_ref, w1_ref, b1_ref, w2_ref, b2_ref, out_ref):
        hi = lax.Precision.HIGHEST
        counts = jnp.sum(parts_ref[...].astype(jnp.float32),
                         axis=0, keepdims=True)
        e = lax.dot_general(
            counts, emb_ref[...], (((1,), (0,)), ((), ())),
            precision=hi, preferred_element_type=jnp.float32) * inv_l
        h = lax.dot_general(
            e, w1_ref[...], (((1,), (1,)), ((), ())),
            preferred_element_type=jnp.float32) + b1_ref[...]
        h = jnp.maximum(h, 0.0)
        v = lax.dot_general(
            h, w2_ref[...], (((1,), (1,)), ((), ())),
            preferred_element_type=jnp.float32) + b2_ref[...]
        out_ref[...] = v

    return pl.pallas_call(
        body,
        out_shape=jax.ShapeDtypeStruct((1, 8), jnp.float32),
    )(parts, emb, W1, b1.reshape(1, -1), W2, b2.reshape(1, -1))


def kernel(ids, emb, W1, b1, W2, b2):
    num_ids = ids.shape[0]
    parts = _make_sc_histogram(num_ids)(ids.astype(jnp.int32))
    out = _tc_head(parts, emb, W1, b1, W2, b2, 1.0 / num_ids)
    return out.reshape(8)


# speedup vs baseline: 1.3629x; 1.0039x over previous
import dataclasses
import functools

import jax
import jax.numpy as jnp
from jax import lax
from jax.experimental import pallas as pl
from jax.experimental.pallas import tpu as pltpu
from jax.experimental.pallas import tpu_sc as plsc

_NUM_CORES = 1
_NUM_SUBCORES = 16
_NUM_TILES = _NUM_CORES * _NUM_SUBCORES
_LANES = 16
_BINS = 256


def _sc_compiler_params():
    cp = pltpu.CompilerParams()
    if "needs_layout_passes" in pltpu.CompilerParams.__dataclass_fields__:
        cp = dataclasses.replace(cp, needs_layout_passes=False)
    return cp


def _make_sc_histogram(num_ids: int):
    per_tile = num_ids // _NUM_TILES
    mesh = plsc.VectorSubcoreMesh(
        core_axis_name="c", subcore_axis_name="s",
        num_cores=_NUM_CORES, num_subcores=_NUM_SUBCORES)

    @functools.partial(
        pl.kernel,
        out_type=jax.ShapeDtypeStruct((_NUM_TILES, _BINS), jnp.int32),
        mesh=mesh,
        scratch_types=[
            pltpu.VMEM((per_tile,), jnp.int32),
            pltpu.VMEM((_BINS,), jnp.int32),
        ],
        compiler_params=_sc_compiler_params(),
    )
    def sc_histogram(ids_hbm, out_hbm, ids_v, hist_v):
        wid = lax.axis_index("s") * _NUM_CORES + lax.axis_index("c")
        pltpu.sync_copy(ids_hbm.at[pl.ds(wid * per_tile, per_tile)], ids_v)

        zeros = jnp.zeros((_LANES,), jnp.int32)

        @pl.loop(0, _BINS, step=_LANES)
        def _(i):
            hist_v[pl.ds(i, _LANES)] = zeros

        @pl.loop(0, per_tile, step=_LANES)
        def _(i):
            ids_vec = ids_v[pl.ds(i, _LANES)]
            counts, last = plsc.scan_count(ids_vec)
            plsc.addupdate_scatter(hist_v, [ids_vec], counts, mask=last)

        pltpu.sync_copy(hist_v, out_hbm.at[wid])

    return sc_histogram


def _tc_head(parts, emb, W1, b1, W2, b2, inv_l):
    def body(parts_ref, emb_ref, w1_ref, b1_ref, w2_ref, b2_ref, out_ref):
        hi = lax.Precision.HIGHEST
        counts = jnp.sum(parts_ref[...].astype(jnp.float32),
                         axis=0, keepdims=True)
        e = lax.dot_general(
            counts, emb_ref[...], (((1,), (0,)), ((), ())),
            precision=hi, preferred_element_type=jnp.float32) * inv_l
        h = lax.dot_general(
            e, w1_ref[...], (((1,), (1,)), ((), ())),
            preferred_element_type=jnp.float32) + b1_ref[...][None, :]
        h = jnp.maximum(h, 0.0)
        v = lax.dot_general(
            h, w2_ref[...], (((1,), (1,)), ((), ())),
            preferred_element_type=jnp.float32) + b2_ref[...][None, :]
        out_ref[...] = v[0]

    return pl.pallas_call(
        body,
        out_shape=jax.ShapeDtypeStruct((8,), jnp.float32),
    )(parts, emb, W1, b1, W2, b2)


def kernel(ids, emb, W1, b1, W2, b2):
    num_ids = ids.shape[0]
    parts = _make_sc_histogram(num_ids)(ids.astype(jnp.int32))
    return _tc_head(parts, emb, W1, b1, W2, b2, 1.0 / num_ids)
